# Initial kernel scaffold; baseline (speedup 1.0000x reference)
import jax
import jax.numpy as jnp
from jax.experimental import pallas as pl


def kernel(v):
    raise NotImplementedError("write your pallas kernel here")


# speedup vs baseline: 16.7170x; 16.7170x over previous
import functools
import jax
import jax.numpy as jnp
from jax.experimental import pallas as pl


def _swd_block(v_ref, o_ref, *, lanes, n_rows):
    half = n_rows // 2
    c = pl.program_id(1)
    x = v_ref[0]
    y = x.reshape(half, 2, lanes)
    mn = jnp.min(y, axis=1)
    mx = jnp.max(y, axis=1)

    lane = jax.lax.broadcasted_iota(jnp.int32, (1, lanes), 1)
    gcol = c * lanes + lane
    shift = jnp.where(gcol == 0, 0, (gcol - 1) % half)

    nbits = max(1, (half - 1).bit_length())
    for k in range(nbits):
        amt = 1 << k
        bit = ((shift >> k) & 1) == 1
        mn = jnp.where(bit, jnp.roll(mn, amt, axis=0), mn)
        mx = jnp.where(bit, jnp.roll(mx, amt, axis=0), mx)

    out = jnp.stack([mn, mx], axis=1).reshape(n_rows, lanes)
    o_ref[0] = out


def kernel(v, interpret=False):
    b, n, d = v.shape
    lanes = min(128, d)
    grid = (b, d // lanes)
    body = functools.partial(_swd_block, lanes=lanes, n_rows=n)
    return pl.pallas_call(
        body,
        grid=grid,
        in_specs=[pl.BlockSpec((1, n, lanes), lambda i, j: (i, 0, j))],
        out_specs=pl.BlockSpec((1, n, lanes), lambda i, j: (i, 0, j)),
        out_shape=jax.ShapeDtypeStruct((b, n, d), v.dtype),
        interpret=interpret,
    )(v)


# speedup vs baseline: 43.6603x; 2.6117x over previous
import functools
import jax
import jax.numpy as jnp
from jax.experimental import pallas as pl


def _swd_block(v_ref, o_ref, *, lanes, n_rows):
    c = pl.program_id(1)
    x = v_ref[0]
    xd = jnp.roll(x, -1, axis=0)
    xu = jnp.roll(x, 1, axis=0)
    row = jax.lax.broadcasted_iota(jnp.int32, (n_rows, 1), 0)
    even = (row & 1) == 0
    z0 = jnp.where(even, jnp.minimum(x, xd), jnp.maximum(x, xu))

    z = jnp.roll(z0, -2, axis=0)
    lane = jax.lax.broadcasted_iota(jnp.int32, (1, lanes), 1)
    nbits = max(1, (lanes - 1).bit_length())
    for k in range(nbits):
        amt = 2 << k
        bit = ((lane >> k) & 1) == 1
        z = jnp.where(bit, jnp.roll(z, amt, axis=0), z)

    gcol0 = (c * lanes + lane) == 0
    z = jnp.where(gcol0, z0, z)

    ch = 2 * lanes
    base = 2 * lanes * c
    for p in range(0, n_rows, ch):
        row0 = (p + base) % n_rows
        o_ref[0, pl.ds(row0, ch), :] = z[p:p + ch, :]


def kernel(v, interpret=False):
    b, n, d = v.shape
    lanes = min(128, d)
    grid = (b, d // lanes)
    body = functools.partial(_swd_block, lanes=lanes, n_rows=n)
    return pl.pallas_call(
        body,
        grid=grid,
        in_specs=[pl.BlockSpec((1, n, lanes), lambda i, j: (i, 0, j))],
        out_specs=pl.BlockSpec((1, n, lanes), lambda i, j: (i, 0, j)),
        out_shape=jax.ShapeDtypeStruct((b, n, d), v.dtype),
        interpret=interpret,
    )(v)
